# baseline (speedup 1.0000x reference)
import functools

import jax
import jax.numpy as jnp
from jax import lax
from jax.experimental import pallas as pl
from jax.experimental.pallas import tpu as pltpu
from jax.experimental.pallas import tpu_sc as plsc

NC = 2
NS = 16
NW = NC * NS

CHUNK = 1024


def _sc_gather(idx_perm, table, b_total, embed_dim):
    b_per_w = b_total // NW
    n_chunks = b_per_w // CHUNK
    pack = 128 // embed_dim
    cq = CHUNK // pack
    mesh = plsc.VectorSubcoreMesh(core_axis_name="c", subcore_axis_name="s")

    @functools.partial(
        pl.kernel,
        mesh=mesh,
        out_type=jax.ShapeDtypeStruct((b_total // pack, 128), jnp.float32),
        scratch_types=[
            pltpu.VMEM((CHUNK,), jnp.int32),
            *[pltpu.VMEM((cq, embed_dim), jnp.float32) for _ in range(pack)],
            pltpu.SemaphoreType.DMA,
        ],
        compiler_params=pltpu.CompilerParams(use_tc_tiling_on_sc=False),
    )
    def k(table_hbm, idx_hbm, out_hbm, idx_v, *rest):
        rows_q = rest[:-1]
        sem = rest[-1]
        wid = lax.axis_index("s") * NC + lax.axis_index("c")
        base = wid * b_per_w

        @pl.loop(0, n_chunks)
        def _(g):
            off = base + g * CHUNK
            pltpu.sync_copy(idx_hbm.at[pl.ds(off, CHUNK)], idx_v)
            copies = [
                pltpu.async_copy(
                    table_hbm.at[idx_v.at[pl.ds(q * cq, cq)]],
                    rows_q[q],
                    sem,
                )
                for q in range(pack)
            ]
            off_p = pl.multiple_of(off // pack, cq)
            for q in range(pack):
                copies[q].wait()
                pltpu.sync_copy(
                    rows_q[q],
                    out_hbm.at[
                        pl.ds(off_p, cq),
                        pl.ds(q * embed_dim, embed_dim),
                    ],
                )

    return k(table, idx_perm)


def kernel(x, table):
    batch, hist = x.shape
    vocab, embed_dim = table.shape
    b_total = batch * hist
    pack = 128 // embed_dim
    idx_flat = x.reshape(b_total).astype(jnp.int32)
    idx_perm = (
        idx_flat.reshape(b_total // CHUNK, CHUNK // pack, pack)
        .transpose(0, 2, 1)
        .reshape(b_total)
    )
    out = _sc_gather(idx_perm, table, b_total, embed_dim)
    return out.reshape(batch, hist, embed_dim)


# speedup vs baseline: 1.0352x; 1.0352x over previous
import functools

import jax
import jax.numpy as jnp
from jax import lax
from jax.experimental import pallas as pl
from jax.experimental.pallas import tpu as pltpu
from jax.experimental.pallas import tpu_sc as plsc

NC = 2
NS = 16
NW = NC * NS

RB = 4


def _sc_gather(x, table, batch, hist, embed_dim):
    rows_per_w = batch // NW
    n_blocks = rows_per_w // RB
    cp = RB * hist
    mesh = plsc.VectorSubcoreMesh(core_axis_name="c", subcore_axis_name="s")

    @functools.partial(
        pl.kernel,
        mesh=mesh,
        out_type=jax.ShapeDtypeStruct((batch, hist, embed_dim), jnp.float32),
        scratch_types=[
            pltpu.VMEM((cp,), jnp.int32),
            pltpu.VMEM((cp, embed_dim), jnp.float32),
            pltpu.SemaphoreType.DMA,
        ],
        compiler_params=pltpu.CompilerParams(use_tc_tiling_on_sc=False),
    )
    def k(table_hbm, x_hbm, out_hbm, idx_v, rows_v, sem):
        wid = lax.axis_index("s") * NC + lax.axis_index("c")
        base = wid * rows_per_w

        @pl.loop(0, n_blocks)
        def _(g):
            b0 = base + g * RB
            for j in range(RB):
                pltpu.sync_copy(
                    x_hbm.at[b0 + j], idx_v.at[pl.ds(j * hist, hist)]
                )
            pltpu.async_copy(table_hbm.at[idx_v], rows_v, sem).wait()
            for j in range(RB):
                pltpu.sync_copy(
                    rows_v.at[pl.ds(j * hist, hist)], out_hbm.at[b0 + j]
                )

    return k(table, x)


def kernel(x, table):
    batch, hist = x.shape
    vocab, embed_dim = table.shape
    return _sc_gather(x.astype(jnp.int32), table, batch, hist, embed_dim)


# speedup vs baseline: 1.1796x; 1.1395x over previous
import functools

import jax
import jax.numpy as jnp
from jax import lax
from jax.experimental import pallas as pl
from jax.experimental.pallas import tpu as pltpu
from jax.experimental.pallas import tpu_sc as plsc

NC = 2
NS = 16
NW = NC * NS

RB = 8


def _sc_gather(x, table, batch, hist, embed_dim):
    rows_per_w = batch // NW
    n_blocks = rows_per_w // RB
    assert n_blocks % 2 == 0 and n_blocks >= 4
    cp = RB * hist
    mesh = plsc.VectorSubcoreMesh(core_axis_name="c", subcore_axis_name="s")

    @functools.partial(
        pl.kernel,
        mesh=mesh,
        out_type=jax.ShapeDtypeStruct((batch, hist, embed_dim), jnp.float32),
        scratch_types=[
            *[pltpu.VMEM((cp,), jnp.int32) for _ in range(2)],
            *[pltpu.VMEM((cp, embed_dim), jnp.float32) for _ in range(2)],
            *[pltpu.SemaphoreType.DMA for _ in range(6)],
        ],
        compiler_params=pltpu.CompilerParams(use_tc_tiling_on_sc=False),
    )
    def k(table_hbm, x_hbm, out_hbm, i0, i1, r0, r1, si0, si1, sg0, sg1,
          so0, so1):
        idx_v = (i0, i1)
        rows_v = (r0, r1)
        sem_i = (si0, si1)
        sem_g = (sg0, sg1)
        sem_o = (so0, so1)
        wid = lax.axis_index("s") * NC + lax.axis_index("c")
        base = wid * rows_per_w

        def idx_start(t, b):
            b0 = base + t * RB
            for j in range(RB):
                pltpu.async_copy(
                    x_hbm.at[b0 + j],
                    idx_v[b].at[pl.ds(j * hist, hist)],
                    sem_i[b],
                )

        def idx_wait(b):
            for j in range(RB):
                pltpu.make_async_copy(
                    x_hbm.at[base],
                    idx_v[b].at[pl.ds(j * hist, hist)],
                    sem_i[b],
                ).wait()

        def gather_start(b):
            pltpu.async_copy(table_hbm.at[idx_v[b]], rows_v[b], sem_g[b])

        def gather_wait(b):
            pltpu.make_async_copy(
                table_hbm.at[idx_v[b]], rows_v[b], sem_g[b]
            ).wait()

        def out_start(t, b):
            b0 = base + t * RB
            for j in range(RB):
                pltpu.async_copy(
                    rows_v[b].at[pl.ds(j * hist, hist)],
                    out_hbm.at[b0 + j],
                    sem_o[b],
                )

        def out_wait(b):
            for j in range(RB):
                pltpu.make_async_copy(
                    rows_v[b].at[pl.ds(j * hist, hist)],
                    out_hbm.at[base],
                    sem_o[b],
                ).wait()

        idx_start(0, 0)
        idx_start(1, 1)
        idx_wait(0)
        gather_start(0)
        idx_wait(1)
        gather_start(1)
        gather_wait(0)
        out_start(0, 0)

        @pl.loop(1, n_blocks // 2)
        def _(p):
            t0 = 2 * p
            out_wait(0)
            idx_start(t0, 0)
            idx_wait(0)
            gather_start(0)
            gather_wait(1)
            out_start(t0 - 1, 1)
            idx_start(t0 + 1, 1)
            idx_wait(1)
            out_wait(1)
            gather_start(1)
            gather_wait(0)
            out_start(t0, 0)

        gather_wait(1)
        out_start(n_blocks - 1, 1)
        out_wait(0)
        out_wait(1)

    return k(table, x)


def kernel(x, table):
    batch, hist = x.shape
    vocab, embed_dim = table.shape
    return _sc_gather(x.astype(jnp.int32), table, batch, hist, embed_dim)


# speedup vs baseline: 1.2554x; 1.0643x over previous
import functools

import jax
import jax.numpy as jnp
from jax import lax
from jax.experimental import pallas as pl
from jax.experimental.pallas import tpu as pltpu
from jax.experimental.pallas import tpu_sc as plsc

NC = 2
NS = 16
NW = NC * NS

RB = 8


def _sc_gather(x, table, batch, hist, embed_dim):
    rows_per_w = batch // NW
    n_blocks = rows_per_w // RB
    assert n_blocks % 2 == 0 and n_blocks >= 4
    cp = RB * hist
    mesh = plsc.VectorSubcoreMesh(core_axis_name="c", subcore_axis_name="s")

    @functools.partial(
        pl.kernel,
        mesh=mesh,
        out_type=jax.ShapeDtypeStruct((batch, hist, embed_dim), jnp.float32),
        scratch_types=[
            *[pltpu.VMEM((cp,), jnp.int32) for _ in range(2)],
            *[pltpu.VMEM((cp, embed_dim), jnp.float32) for _ in range(2)],
            *[pltpu.SemaphoreType.DMA for _ in range(6)],
        ],
        compiler_params=pltpu.CompilerParams(use_tc_tiling_on_sc=False),
    )
    def k(table_hbm, x_hbm, out_hbm, i0, i1, r0, r1, si0, si1, sg0, sg1,
          so0, so1):
        idx_v = (i0, i1)
        rows_v = (r0, r1)
        sem_i = (si0, si1)
        sem_g = (sg0, sg1)
        sem_o = (so0, so1)
        wid = lax.axis_index("s") * NC + lax.axis_index("c")
        base = wid * rows_per_w

        def idx_start(t, b):
            b0 = base + t * RB
            for j in range(RB):
                pltpu.async_copy(
                    x_hbm.at[b0 + j],
                    idx_v[b].at[pl.ds(j * hist, hist)],
                    sem_i[b],
                )

        def idx_wait(b):
            for j in range(RB):
                pltpu.make_async_copy(
                    x_hbm.at[base],
                    idx_v[b].at[pl.ds(j * hist, hist)],
                    sem_i[b],
                ).wait()

        def gather_start(b):
            pltpu.async_copy(table_hbm.at[idx_v[b]], rows_v[b], sem_g[b])

        def gather_wait(b):
            pltpu.make_async_copy(
                table_hbm.at[idx_v[b]], rows_v[b], sem_g[b]
            ).wait()

        def out_start(t, b):
            b0 = base + t * RB
            for j in range(RB):
                pltpu.async_copy(
                    rows_v[b].at[pl.ds(j * hist, hist)],
                    out_hbm.at[b0 + j],
                    sem_o[b],
                )

        def out_wait(b):
            for j in range(RB):
                pltpu.make_async_copy(
                    rows_v[b].at[pl.ds(j * hist, hist)],
                    out_hbm.at[base],
                    sem_o[b],
                ).wait()

        idx_start(0, 0)
        idx_start(1, 1)
        idx_wait(0)
        gather_start(0)
        idx_wait(1)
        gather_start(1)
        gather_wait(0)
        out_start(0, 0)

        @pl.loop(1, n_blocks // 2)
        def _(p):
            t0 = 2 * p
            out_wait(0)
            idx_start(t0, 0)
            idx_wait(0)
            gather_start(0)
            gather_wait(1)
            out_start(t0 - 1, 1)
            idx_start(t0 + 1, 1)
            idx_wait(1)
            out_wait(1)
            gather_start(1)
            gather_wait(0)
            out_start(t0, 0)

        gather_wait(1)
        out_start(n_blocks - 1, 1)
        out_wait(0)
        out_wait(1)

    return k(table, x)


N_SLICES = 4


def kernel(x, table):
    batch, hist = x.shape
    vocab, embed_dim = table.shape
    xi = x.astype(jnp.int32)
    bs = batch // N_SLICES
    outs = [
        _sc_gather(
            lax.slice(xi, (i * bs, 0), ((i + 1) * bs, hist)),
            table,
            bs,
            hist,
            embed_dim,
        )
        for i in range(N_SLICES)
    ]
    return jnp.concatenate(outs, axis=0)


# speedup vs baseline: 1.2889x; 1.0267x over previous
import functools

import jax
import jax.numpy as jnp
from jax import lax
from jax.experimental import pallas as pl
from jax.experimental.pallas import tpu as pltpu
from jax.experimental.pallas import tpu_sc as plsc

NC = 2
NS = 16
NW = NC * NS

RB = 32


def _sc_gather(x, table, batch, hist, embed_dim):
    rows_per_w = batch // NW
    n_blocks = rows_per_w // RB
    assert n_blocks % 2 == 0 and n_blocks >= 4
    cp = RB * hist
    mesh = plsc.VectorSubcoreMesh(core_axis_name="c", subcore_axis_name="s")

    @functools.partial(
        pl.kernel,
        mesh=mesh,
        out_type=jax.ShapeDtypeStruct((batch, hist, embed_dim), jnp.float32),
        scratch_types=[
            *[pltpu.VMEM((cp,), jnp.int32) for _ in range(2)],
            *[pltpu.VMEM((cp, embed_dim), jnp.float32) for _ in range(2)],
            *[pltpu.SemaphoreType.DMA for _ in range(6)],
        ],
        compiler_params=pltpu.CompilerParams(use_tc_tiling_on_sc=False),
    )
    def k(table_hbm, x_hbm, out_hbm, i0, i1, r0, r1, si0, si1, sg0, sg1,
          so0, so1):
        idx_v = (i0, i1)
        rows_v = (r0, r1)
        sem_i = (si0, si1)
        sem_g = (sg0, sg1)
        sem_o = (so0, so1)
        wid = lax.axis_index("s") * NC + lax.axis_index("c")
        base = wid * rows_per_w

        def idx_start(t, b):
            b0 = base + t * RB
            for j in range(RB):
                pltpu.async_copy(
                    x_hbm.at[b0 + j],
                    idx_v[b].at[pl.ds(j * hist, hist)],
                    sem_i[b],
                )

        def idx_wait(b):
            for j in range(RB):
                pltpu.make_async_copy(
                    x_hbm.at[base],
                    idx_v[b].at[pl.ds(j * hist, hist)],
                    sem_i[b],
                ).wait()

        def gather_start(b):
            pltpu.async_copy(table_hbm.at[idx_v[b]], rows_v[b], sem_g[b])

        def gather_wait(b):
            pltpu.make_async_copy(
                table_hbm.at[idx_v[b]], rows_v[b], sem_g[b]
            ).wait()

        def out_start(t, b):
            b0 = base + t * RB
            for j in range(RB):
                pltpu.async_copy(
                    rows_v[b].at[pl.ds(j * hist, hist)],
                    out_hbm.at[b0 + j],
                    sem_o[b],
                )

        def out_wait(b):
            for j in range(RB):
                pltpu.make_async_copy(
                    rows_v[b].at[pl.ds(j * hist, hist)],
                    out_hbm.at[base],
                    sem_o[b],
                ).wait()

        idx_start(0, 0)
        idx_start(1, 1)
        idx_wait(0)
        gather_start(0)
        idx_wait(1)
        gather_start(1)
        gather_wait(0)
        out_start(0, 0)

        @pl.loop(1, n_blocks // 2)
        def _(p):
            t0 = 2 * p
            out_wait(0)
            idx_start(t0, 0)
            idx_wait(0)
            gather_start(0)
            gather_wait(1)
            out_start(t0 - 1, 1)
            idx_start(t0 + 1, 1)
            idx_wait(1)
            out_wait(1)
            gather_start(1)
            gather_wait(0)
            out_start(t0, 0)

        gather_wait(1)
        out_start(n_blocks - 1, 1)
        out_wait(0)
        out_wait(1)

    return k(table, x)


N_SLICES = 5


def kernel(x, table):
    batch, hist = x.shape
    vocab, embed_dim = table.shape
    xi = x.astype(jnp.int32)
    hs = hist // N_SLICES
    outs = [
        _sc_gather(
            lax.slice(xi, (0, i * hs), (batch, (i + 1) * hs)),
            table,
            batch,
            hs,
            embed_dim,
        )
        for i in range(N_SLICES)
    ]
    return jnp.concatenate(outs, axis=1)
